# Initial kernel scaffold; baseline (speedup 1.0000x reference)
import jax
import jax.numpy as jnp
from jax.experimental import pallas as pl


def kernel(edge_index, h, r, norm, W):
    raise NotImplementedError("write your pallas kernel here")


# speedup vs baseline: 13.4426x; 13.4426x over previous
import functools

import jax
import jax.numpy as jnp
from jax import lax
from jax.experimental import pallas as pl
from jax.experimental.pallas import tpu as pltpu
from jax.experimental.pallas import tpu_sc as plsc

N = 10000
D = 128
R = 8
E = 320000

NC = 2
NS = 16
NW = NC * NS
E_PER_W = E // NW
CHUNK = 80
NCHUNK = E_PER_W // CHUNK
ZROWS = 16
S_STRIDE = 624
S_ROWS = 640


def _proj_body(h_ref, w_ref, out_ref):
    out_ref[0] = jnp.dot(h_ref[...], w_ref[0],
                         preferred_element_type=jnp.float32)


def _project(h, W):
    BLK = 400
    return pl.pallas_call(
        _proj_body,
        grid=(R, N // BLK),
        in_specs=[
            pl.BlockSpec((BLK, D), lambda ri, bi: (bi, 0)),
            pl.BlockSpec((1, D, D), lambda ri, bi: (ri, 0, 0)),
        ],
        out_specs=pl.BlockSpec((1, BLK, D), lambda ri, bi: (ri, bi, 0)),
        out_shape=jax.ShapeDtypeStruct((R, N, D), jnp.float32),
    )(h, W)


def _idx_body(src_ref, r_ref, out_ref):
    out_ref[...] = r_ref[...] * N + src_ref[...]


def _edge_idx(src2, r2):
    return pl.pallas_call(
        _idx_body,
        out_shape=jax.ShapeDtypeStruct((E // 128, 128), jnp.int32),
    )(src2, r2)


def _sc_edge_body(idx_hbm, dst_hbm, norm_hbm, hall_hbm, out_hbm,
                  idx_v, dst_v, norm_v, rows_v, zero_v, agg_sh, sem):
    cid = lax.axis_index("c")
    sid = lax.axis_index("s")
    wid = cid * NS + sid
    base = wid * E_PER_W

    pltpu.sync_copy(idx_hbm.at[pl.ds(base, E_PER_W)], idx_v)
    pltpu.sync_copy(dst_hbm.at[wid], dst_v)
    pltpu.sync_copy(norm_hbm.at[pl.ds(base, E_PER_W)], norm_v)

    def zero_body(i, carry):
        for c in range(D // 16):
            zero_v[i, pl.ds(c * 16, 16)] = jnp.zeros((16,), jnp.float32)
        return carry
    lax.fori_loop(0, ZROWS, zero_body, 0)

    def zcopy_body(j, carry):
        pltpu.sync_copy(
            zero_v, agg_sh.at[pl.ds(sid * S_STRIDE + j * ZROWS, ZROWS)])
        return carry
    lax.fori_loop(0, S_ROWS // ZROWS, zcopy_body, 0)
    plsc.subcore_barrier()

    def chunk_body(t, carry):
        off = t * CHUNK
        pltpu.async_copy(
            hall_hbm.at[idx_v.at[pl.ds(off, CHUNK)]], rows_v, sem).wait()

        def group_body(g, c2):
            nv = norm_v[pl.ds(off + g * 16, 16)]
            for k in range(16):
                nb = nv[k]
                e = g * 16 + k
                for c in range(D // 16):
                    rows_v[e, pl.ds(c * 16, 16)] = (
                        rows_v[e, pl.ds(c * 16, 16)] * nb)
            return c2
        lax.fori_loop(0, CHUNK // 16, group_body, 0)

        pltpu.sync_copy(rows_v, agg_sh.at[dst_v.at[t]], add=True)
        return carry
    lax.fori_loop(0, NCHUNK, chunk_body, 0)

    plsc.subcore_barrier()
    pltpu.sync_copy(
        agg_sh.at[pl.ds(sid * S_STRIDE, S_ROWS)],
        out_hbm.at[cid, pl.ds(sid * S_STRIDE, S_ROWS)])


def _sc_edges(idx, dst3, norm_flat, h_all):
    mesh = plsc.VectorSubcoreMesh(core_axis_name="c", subcore_axis_name="s")
    fn = functools.partial(
        pl.kernel, mesh=mesh,
        out_type=jax.ShapeDtypeStruct((NC, N, D), jnp.float32),
        scratch_types=[
            pltpu.VMEM((E_PER_W,), jnp.int32),
            pltpu.VMEM((NCHUNK, CHUNK), jnp.int32),
            pltpu.VMEM((E_PER_W,), jnp.float32),
            pltpu.VMEM((CHUNK, D), jnp.float32),
            pltpu.VMEM((ZROWS, D), jnp.float32),
            pltpu.VMEM_SHARED((N, D), jnp.float32),
            pltpu.SemaphoreType.DMA,
        ],
    )(_sc_edge_body)
    return fn(idx, dst3, norm_flat, h_all)


def _combine_body(p_ref, out_ref):
    out_ref[...] = jnp.maximum(p_ref[0] + p_ref[1], 0.0)


def _combine(partials):
    BLK = 400
    return pl.pallas_call(
        _combine_body,
        grid=(N // BLK,),
        in_specs=[pl.BlockSpec((NC, BLK, D), lambda bi: (0, bi, 0))],
        out_specs=pl.BlockSpec((BLK, D), lambda bi: (bi, 0)),
        out_shape=jax.ShapeDtypeStruct((N, D), jnp.float32),
    )(partials)


def kernel(edge_index, h, r, norm, W):
    src2 = edge_index[0].reshape(E // 128, 128)
    r2 = r.reshape(E // 128, 128)
    dst3 = edge_index[1].reshape(NW, NCHUNK, CHUNK)
    norm_flat = norm.reshape(E)
    idx = _edge_idx(src2, r2).reshape(E)
    h_all = _project(h, W).reshape(R * N, D)
    partials = _sc_edges(idx, dst3, norm_flat, h_all)
    return _combine(partials)


# speedup vs baseline: 22.4019x; 1.6665x over previous
import functools

import jax
import jax.numpy as jnp
from jax import lax
from jax.experimental import pallas as pl
from jax.experimental.pallas import tpu as pltpu
from jax.experimental.pallas import tpu_sc as plsc

N = 10000
D = 128
R = 8
E = 320000

NC = 2
NS = 16
NW = NC * NS
E_PER_W = E // NW
CHUNK = 80
NCHUNK = E_PER_W // CHUNK
NPAIR = (NCHUNK - 1) // 2
ZROWS = 16
S_STRIDE = 624
S_ROWS = 640


def _proj_body(h_ref, w_ref, out_ref):
    out_ref[...] = jnp.dot(h_ref[...], w_ref[...],
                           preferred_element_type=jnp.float32)


def _project(h, W2):
    BLK = 1000
    return pl.pallas_call(
        _proj_body,
        grid=(N // BLK,),
        in_specs=[
            pl.BlockSpec((BLK, D), lambda bi: (bi, 0)),
            pl.BlockSpec((D, R * D), lambda bi: (0, 0)),
        ],
        out_specs=pl.BlockSpec((BLK, R * D), lambda bi: (bi, 0)),
        out_shape=jax.ShapeDtypeStruct((N, R * D), jnp.float32),
    )(h, W2)


def _idx_body(src_ref, r_ref, out_ref):
    out_ref[...] = src_ref[...] * R + r_ref[...]


def _edge_idx(src2, r2):
    return pl.pallas_call(
        _idx_body,
        out_shape=jax.ShapeDtypeStruct((E // 128, 128), jnp.int32),
    )(src2, r2)


def _sc_edge_body(idx_hbm, dst_hbm, norm_hbm, hall_hbm, out_hbm,
                  idx_v, dst_v, rows0, rows1, n0, n1, zero_v, agg_sh,
                  gsem0, gsem1):
    cid = lax.axis_index("c")
    sid = lax.axis_index("s")
    wid = cid * NS + sid
    base = wid * E_PER_W

    pltpu.sync_copy(idx_hbm.at[pl.ds(base, E_PER_W)], idx_v)
    pltpu.sync_copy(dst_hbm.at[wid], dst_v)

    def zero_body(i, carry):
        for c in range(D // 16):
            zero_v[i, pl.ds(c * 16, 16)] = jnp.zeros((16,), jnp.float32)
        return carry
    lax.fori_loop(0, ZROWS, zero_body, 0)

    def zcopy_body(j, carry):
        pltpu.sync_copy(
            zero_v, agg_sh.at[pl.ds(sid * S_STRIDE + j * ZROWS, ZROWS)])
        return carry
    lax.fori_loop(0, S_ROWS // ZROWS, zcopy_body, 0)
    plsc.subcore_barrier()

    def issue(t, rows_ref, nrm_ref, sem):
        off = t * CHUNK
        pltpu.async_copy(
            hall_hbm.at[idx_v.at[pl.ds(off, CHUNK)]], rows_ref, sem)
        pltpu.async_copy(
            norm_hbm.at[pl.ds(base + off, CHUNK)], nrm_ref, sem)

    def wait(t, rows_ref, nrm_ref, sem):
        off = t * CHUNK
        pltpu.make_async_copy(
            hall_hbm.at[idx_v.at[pl.ds(off, CHUNK)]], rows_ref, sem).wait()
        pltpu.make_async_copy(
            norm_hbm.at[pl.ds(base + off, CHUNK)], nrm_ref, sem).wait()

    def scale(rows_ref, nrm_ref):
        def group_body(g, c2):
            nv = nrm_ref[pl.ds(g * 16, 16)]
            for k in range(16):
                nb = nv[k]
                e = g * 16 + k
                for c in range(D // 16):
                    rows_ref[e, pl.ds(c * 16, 16)] = (
                        rows_ref[e, pl.ds(c * 16, 16)] * nb)
            return c2
        lax.fori_loop(0, CHUNK // 16, group_body, 0)

    issue(0, rows0, n0, gsem0)

    def pair_body(i, carry):
        t0 = 2 * i
        t1 = t0 + 1
        issue(t1, rows1, n1, gsem1)
        wait(t0, rows0, n0, gsem0)
        scale(rows0, n0)
        pltpu.sync_copy(rows0, agg_sh.at[dst_v.at[t0]], add=True)
        issue(t0 + 2, rows0, n0, gsem0)
        wait(t1, rows1, n1, gsem1)
        scale(rows1, n1)
        pltpu.sync_copy(rows1, agg_sh.at[dst_v.at[t1]], add=True)
        return carry
    lax.fori_loop(0, NPAIR, pair_body, 0)

    wait(NCHUNK - 1, rows0, n0, gsem0)
    scale(rows0, n0)
    pltpu.sync_copy(rows0, agg_sh.at[dst_v.at[NCHUNK - 1]], add=True)

    plsc.subcore_barrier()
    pltpu.sync_copy(
        agg_sh.at[pl.ds(sid * S_STRIDE, S_ROWS)],
        out_hbm.at[cid, pl.ds(sid * S_STRIDE, S_ROWS)])


def _sc_edges(idx, dst3, norm_flat, h_all):
    mesh = plsc.VectorSubcoreMesh(core_axis_name="c", subcore_axis_name="s")
    fn = functools.partial(
        pl.kernel, mesh=mesh,
        out_type=jax.ShapeDtypeStruct((NC, N, D), jnp.float32),
        scratch_types=[
            pltpu.VMEM((E_PER_W,), jnp.int32),
            pltpu.VMEM((NCHUNK, CHUNK), jnp.int32),
            pltpu.VMEM((CHUNK, D), jnp.float32),
            pltpu.VMEM((CHUNK, D), jnp.float32),
            pltpu.VMEM((CHUNK,), jnp.float32),
            pltpu.VMEM((CHUNK,), jnp.float32),
            pltpu.VMEM((ZROWS, D), jnp.float32),
            pltpu.VMEM_SHARED((N, D), jnp.float32),
            pltpu.SemaphoreType.DMA,
            pltpu.SemaphoreType.DMA,
        ],
    )(_sc_edge_body)
    return fn(idx, dst3, norm_flat, h_all)


def _combine_body(p_ref, out_ref):
    out_ref[...] = jnp.maximum(p_ref[0] + p_ref[1], 0.0)


def _combine(partials):
    BLK = 400
    return pl.pallas_call(
        _combine_body,
        grid=(N // BLK,),
        in_specs=[pl.BlockSpec((NC, BLK, D), lambda bi: (0, bi, 0))],
        out_specs=pl.BlockSpec((BLK, D), lambda bi: (bi, 0)),
        out_shape=jax.ShapeDtypeStruct((N, D), jnp.float32),
    )(partials)


def kernel(edge_index, h, r, norm, W):
    src2 = edge_index[0].reshape(E // 128, 128)
    r2 = r.reshape(E // 128, 128)
    dst3 = edge_index[1].reshape(NW, NCHUNK, CHUNK)
    norm_flat = norm.reshape(E)
    W2 = jnp.transpose(W, (1, 0, 2)).reshape(D, R * D)
    idx = _edge_idx(src2, r2).reshape(E)
    h_all = _project(h, W2).reshape(N * R, D)
    partials = _sc_edges(idx, dst3, norm_flat, h_all)
    return _combine(partials)


# speedup vs baseline: 29.4604x; 1.3151x over previous
import functools

import jax
import jax.numpy as jnp
from jax import lax
from jax.experimental import pallas as pl
from jax.experimental.pallas import tpu as pltpu
from jax.experimental.pallas import tpu_sc as plsc

N = 10000
D = 128
R = 8
E = 320000

NC = 2
NS = 16
NW = NC * NS
E_PER_W = E // NW
CHUNK = 80
NCHUNK = E_PER_W // CHUNK
NTRIPLE = (NCHUNK - 5) // 3
ZROWS = 16
S_STRIDE = 624
S_ROWS = 640


def _proj_body(h_ref, w_ref, out_ref):
    for rr in range(R):
        out_ref[rr] = jnp.dot(h_ref[...], w_ref[rr],
                              preferred_element_type=jnp.float32)


def _project(h, W):
    BLK = 1000
    return pl.pallas_call(
        _proj_body,
        grid=(N // BLK,),
        in_specs=[
            pl.BlockSpec((BLK, D), lambda bi: (bi, 0)),
            pl.BlockSpec((R, D, D), lambda bi: (0, 0, 0)),
        ],
        out_specs=pl.BlockSpec((R, BLK, D), lambda bi: (0, bi, 0)),
        out_shape=jax.ShapeDtypeStruct((R, N, D), jnp.float32),
    )(h, W)


def _idx_body(src_ref, r_ref, out_ref):
    out_ref[...] = r_ref[...] * N + src_ref[...]


def _edge_idx(src2, r2):
    return pl.pallas_call(
        _idx_body,
        out_shape=jax.ShapeDtypeStruct((E // 128, 128), jnp.int32),
    )(src2, r2)


def _sc_edge_body(idx_hbm, dst_hbm, norm_hbm, hall_hbm, out_hbm,
                  idx_v, rows, nrm, dstb, zero_v, agg_sh, gsem, ssem):
    cid = lax.axis_index("c")
    sid = lax.axis_index("s")
    wid = cid * NS + sid
    base = wid * E_PER_W

    pltpu.sync_copy(idx_hbm.at[pl.ds(base, E_PER_W)], idx_v)

    def zero_body(i, carry):
        for c in range(D // 16):
            zero_v[i, pl.ds(c * 16, 16)] = jnp.zeros((16,), jnp.float32)
        return carry
    lax.fori_loop(0, ZROWS, zero_body, 0)

    def zcopy_body(j, carry):
        pltpu.sync_copy(
            zero_v, agg_sh.at[pl.ds(sid * S_STRIDE + j * ZROWS, ZROWS)])
        return carry
    lax.fori_loop(0, S_ROWS // ZROWS, zcopy_body, 0)
    plsc.subcore_barrier()

    def issue_g(t, b):
        off = t * CHUNK
        pltpu.async_copy(
            hall_hbm.at[idx_v.at[pl.ds(off, CHUNK)]], rows[b], gsem[b])
        pltpu.async_copy(
            norm_hbm.at[pl.ds(base + off, CHUNK)], nrm[b], gsem[b])
        pltpu.async_copy(
            dst_hbm.at[pl.ds(base + off, CHUNK)], dstb[b], gsem[b])

    def wait_g(t, b):
        off = t * CHUNK
        pltpu.make_async_copy(
            hall_hbm.at[idx_v.at[pl.ds(off, CHUNK)]], rows[b],
            gsem[b]).wait()
        pltpu.make_async_copy(
            norm_hbm.at[pl.ds(base + off, CHUNK)], nrm[b], gsem[b]).wait()
        pltpu.make_async_copy(
            dst_hbm.at[pl.ds(base + off, CHUNK)], dstb[b], gsem[b]).wait()

    def scale(b):
        def group_body(g, c2):
            nv = nrm[b][pl.ds(g * 16, 16)]
            for k in range(16):
                nb = nv[k]
                e = g * 16 + k
                for c in range(D // 16):
                    rows[b][e, pl.ds(c * 16, 16)] = (
                        rows[b][e, pl.ds(c * 16, 16)] * nb)
            return c2
        lax.fori_loop(0, CHUNK // 16, group_body, 0)

    def issue_s(b):
        pltpu.async_copy(rows[b], agg_sh.at[dstb[b]], ssem[b], add=True)

    def wait_s(b):
        pltpu.make_async_copy(rows[b], agg_sh.at[dstb[b]], ssem[b]).wait()

    issue_g(0, 0)
    issue_g(1, 1)
    wait_g(0, 0); scale(0); issue_s(0)
    issue_g(2, 2)
    wait_g(1, 1); scale(1); issue_s(1)
    wait_s(0); issue_g(3, 0)
    wait_g(2, 2); scale(2); issue_s(2)
    wait_s(1); issue_g(4, 1)

    def triple_body(i, carry):
        t = 3 * i
        wait_g(t, 0); scale(0); issue_s(0)
        wait_s(2); issue_g(t + 2, 2)
        wait_g(t + 1, 1); scale(1); issue_s(1)
        wait_s(0); issue_g(t + 3, 0)
        wait_g(t + 2, 2); scale(2); issue_s(2)
        wait_s(1); issue_g(t + 4, 1)
        return carry
    lax.fori_loop(1, NTRIPLE + 1, triple_body, 0)

    wait_g(NCHUNK - 2, 0); scale(0); issue_s(0)
    wait_s(2)
    wait_g(NCHUNK - 1, 1); scale(1); issue_s(1)
    wait_s(0)
    wait_s(1)

    plsc.subcore_barrier()
    pltpu.sync_copy(
        agg_sh.at[pl.ds(sid * S_STRIDE, S_ROWS)],
        out_hbm.at[cid, pl.ds(sid * S_STRIDE, S_ROWS)])


def _sc_edges(idx, dst, norm_flat, h_all):
    mesh = plsc.VectorSubcoreMesh(core_axis_name="c", subcore_axis_name="s")
    fn = functools.partial(
        pl.kernel, mesh=mesh,
        out_type=jax.ShapeDtypeStruct((NC, N, D), jnp.float32),
        scratch_types=[
            pltpu.VMEM((E_PER_W,), jnp.int32),
            [pltpu.VMEM((CHUNK, D), jnp.float32)] * 3,
            [pltpu.VMEM((CHUNK,), jnp.float32)] * 3,
            [pltpu.VMEM((CHUNK,), jnp.int32)] * 3,
            pltpu.VMEM((ZROWS, D), jnp.float32),
            pltpu.VMEM_SHARED((N, D), jnp.float32),
            [pltpu.SemaphoreType.DMA] * 3,
            [pltpu.SemaphoreType.DMA] * 3,
        ],
    )(_sc_edge_body)
    return fn(idx, dst, norm_flat, h_all)


def _combine_body(p_ref, out_ref):
    out_ref[...] = jnp.maximum(p_ref[0] + p_ref[1], 0.0)


def _combine(partials):
    BLK = 400
    return pl.pallas_call(
        _combine_body,
        grid=(N // BLK,),
        in_specs=[pl.BlockSpec((NC, BLK, D), lambda bi: (0, bi, 0))],
        out_specs=pl.BlockSpec((BLK, D), lambda bi: (bi, 0)),
        out_shape=jax.ShapeDtypeStruct((N, D), jnp.float32),
    )(partials)


def kernel(edge_index, h, r, norm, W):
    src2 = edge_index[0].reshape(E // 128, 128)
    r2 = r.reshape(E // 128, 128)
    dst = edge_index[1]
    norm_flat = norm.reshape(E)
    idx = _edge_idx(src2, r2).reshape(E)
    h_all = _project(h, W).reshape(R * N, D)
    partials = _sc_edges(idx, dst, norm_flat, h_all)
    return _combine(partials)


# speedup vs baseline: 30.3721x; 1.0309x over previous
import functools

import jax
import jax.numpy as jnp
from jax import lax
from jax.experimental import pallas as pl
from jax.experimental.pallas import tpu as pltpu
from jax.experimental.pallas import tpu_sc as plsc

N = 10000
D = 128
R = 8
E = 320000

NC = 2
NS = 16
NW = NC * NS
E_PER_W = E // NW
CHUNK = 80
NCHUNK = E_PER_W // CHUNK
NTRIPLE = (NCHUNK - 5) // 3
ZROWS = 16
S_STRIDE = 624
S_ROWS = 640


def _proj_body(h_ref, w_ref, out_ref):
    for rr in range(R):
        out_ref[rr] = jnp.dot(h_ref[...], w_ref[rr],
                              preferred_element_type=jnp.float32)


def _project(h, W):
    BLK = 1000
    return pl.pallas_call(
        _proj_body,
        grid=(N // BLK,),
        in_specs=[
            pl.BlockSpec((BLK, D), lambda bi: (bi, 0)),
            pl.BlockSpec((R, D, D), lambda bi: (0, 0, 0)),
        ],
        out_specs=pl.BlockSpec((R, BLK, D), lambda bi: (0, bi, 0)),
        out_shape=jax.ShapeDtypeStruct((R, N, D), jnp.float32),
    )(h, W)


def _idx_body(ei_ref, r_ref, out_ref):
    out_ref[...] = r_ref[...] * N + ei_ref[0]


def _edge_idx(ei3, r2):
    return pl.pallas_call(
        _idx_body,
        grid=(1,),
        in_specs=[
            pl.BlockSpec((1, E // 128, 128), lambda i: (0, 0, 0)),
            pl.BlockSpec((E // 128, 128), lambda i: (0, 0)),
        ],
        out_specs=pl.BlockSpec((E // 128, 128), lambda i: (0, 0)),
        out_shape=jax.ShapeDtypeStruct((E // 128, 128), jnp.int32),
    )(ei3, r2)


def _sc_edge_body(idx_hbm, ei_hbm, norm_hbm, hall_hbm, out_hbm,
                  idx_v, rows, nrm, dstb, zero_v, agg_sh, gsem, ssem):
    cid = lax.axis_index("c")
    sid = lax.axis_index("s")
    wid = cid * NS + sid
    base = wid * E_PER_W

    pltpu.sync_copy(idx_hbm.at[pl.ds(base, E_PER_W)], idx_v)

    def zero_body(i, carry):
        for c in range(D // 16):
            zero_v[i, pl.ds(c * 16, 16)] = jnp.zeros((16,), jnp.float32)
        return carry
    lax.fori_loop(0, ZROWS, zero_body, 0)

    def zcopy_body(j, carry):
        pltpu.sync_copy(
            zero_v, agg_sh.at[pl.ds(sid * S_STRIDE + j * ZROWS, ZROWS)])
        return carry
    lax.fori_loop(0, S_ROWS // ZROWS, zcopy_body, 0)
    plsc.subcore_barrier()

    def issue_g(t, b):
        off = t * CHUNK
        pltpu.async_copy(
            hall_hbm.at[idx_v.at[pl.ds(off, CHUNK)]], rows[b], gsem[b])
        pltpu.async_copy(
            norm_hbm.at[pl.ds(base + off, CHUNK)], nrm[b], gsem[b])
        pltpu.async_copy(
            ei_hbm.at[pl.ds(E + base + off, CHUNK)], dstb[b], gsem[b])

    def wait_g(t, b):
        off = t * CHUNK
        pltpu.make_async_copy(
            hall_hbm.at[idx_v.at[pl.ds(off, CHUNK)]], rows[b],
            gsem[b]).wait()
        pltpu.make_async_copy(
            norm_hbm.at[pl.ds(base + off, CHUNK)], nrm[b], gsem[b]).wait()
        pltpu.make_async_copy(
            ei_hbm.at[pl.ds(E + base + off, CHUNK)], dstb[b], gsem[b]).wait()

    def scale(b):
        def group_body(g, c2):
            nv = nrm[b][pl.ds(g * 16, 16)]
            for k in range(16):
                nb = nv[k]
                e = g * 16 + k
                for c in range(D // 16):
                    rows[b][e, pl.ds(c * 16, 16)] = (
                        rows[b][e, pl.ds(c * 16, 16)] * nb)
            return c2
        lax.fori_loop(0, CHUNK // 16, group_body, 0)

    def issue_s(b):
        pltpu.async_copy(rows[b], agg_sh.at[dstb[b]], ssem[b], add=True)

    def wait_s(b):
        pltpu.make_async_copy(rows[b], agg_sh.at[dstb[b]], ssem[b]).wait()

    issue_g(0, 0)
    issue_g(1, 1)
    wait_g(0, 0); scale(0); issue_s(0)
    issue_g(2, 2)
    wait_g(1, 1); scale(1); issue_s(1)
    wait_s(0); issue_g(3, 0)
    wait_g(2, 2); scale(2); issue_s(2)
    wait_s(1); issue_g(4, 1)

    def triple_body(i, carry):
        t = 3 * i
        wait_g(t, 0); scale(0); issue_s(0)
        wait_s(2); issue_g(t + 2, 2)
        wait_g(t + 1, 1); scale(1); issue_s(1)
        wait_s(0); issue_g(t + 3, 0)
        wait_g(t + 2, 2); scale(2); issue_s(2)
        wait_s(1); issue_g(t + 4, 1)
        return carry
    lax.fori_loop(1, NTRIPLE + 1, triple_body, 0)

    wait_g(NCHUNK - 2, 0); scale(0); issue_s(0)
    wait_s(2)
    wait_g(NCHUNK - 1, 1); scale(1); issue_s(1)
    wait_s(0)
    wait_s(1)

    plsc.subcore_barrier()
    pltpu.sync_copy(
        agg_sh.at[pl.ds(sid * S_STRIDE, S_ROWS)],
        out_hbm.at[cid, pl.ds(sid * S_STRIDE, S_ROWS)])


def _sc_edges(idx, edge_index, norm_flat, h_all):
    mesh = plsc.VectorSubcoreMesh(core_axis_name="c", subcore_axis_name="s")
    fn = functools.partial(
        pl.kernel, mesh=mesh,
        out_type=jax.ShapeDtypeStruct((NC, N, D), jnp.float32),
        scratch_types=[
            pltpu.VMEM((E_PER_W,), jnp.int32),
            [pltpu.VMEM((CHUNK, D), jnp.float32)] * 3,
            [pltpu.VMEM((CHUNK,), jnp.float32)] * 3,
            [pltpu.VMEM((CHUNK,), jnp.int32)] * 3,
            pltpu.VMEM((ZROWS, D), jnp.float32),
            pltpu.VMEM_SHARED((N, D), jnp.float32),
            [pltpu.SemaphoreType.DMA] * 3,
            [pltpu.SemaphoreType.DMA] * 3,
        ],
    )(_sc_edge_body)
    return fn(idx, edge_index, norm_flat, h_all)


def _combine_body(p_ref, out_ref):
    out_ref[...] = jnp.maximum(p_ref[0] + p_ref[1], 0.0)


def _combine(partials):
    BLK = 400
    return pl.pallas_call(
        _combine_body,
        grid=(N // BLK,),
        in_specs=[pl.BlockSpec((NC, BLK, D), lambda bi: (0, bi, 0))],
        out_specs=pl.BlockSpec((BLK, D), lambda bi: (bi, 0)),
        out_shape=jax.ShapeDtypeStruct((N, D), jnp.float32),
    )(partials)


def kernel(edge_index, h, r, norm, W):
    ei3 = edge_index.reshape(2, E // 128, 128)
    r2 = r.reshape(E // 128, 128)
    norm_flat = norm.reshape(E)
    idx = _edge_idx(ei3, r2).reshape(E)
    h_all = _project(h, W).reshape(R * N, D)
    partials = _sc_edges(idx, edge_index.reshape(2 * E), norm_flat, h_all)
    return _combine(partials)
